# baseline (speedup 1.0000x reference)
import functools

import jax
import jax.numpy as jnp
from jax import lax
from jax.experimental import pallas as pl
from jax.experimental.pallas import tpu as pltpu
from jax.experimental.pallas import tpu_sc as plsc

_NW = 32
_G = 8
_MINOR = 128
_CHG = 4


def _sc_body(n_total_g, n_pos_g, sl, x_hbm, pos_hbm, idx_hbm, out_hbm,
             xbuf, idxbuf, sx0, sx1, sx2, sg0, sg1, sg2, st0, st1, st2):
    cid = lax.axis_index("c")
    sid = lax.axis_index("s")
    wid = sid * 2 + cid
    g_per_w = n_total_g // _NW
    n_chunks = g_per_w // _CHG
    g0 = wid * g_per_w

    sem_x = [sx0, sx1, sx2]
    sem_g = [sg0, sg1, sg2]
    sem_st = [st0, st1, st2]

    i0 = pl.multiple_of(wid * n_chunks, 8)
    pltpu.sync_copy(idx_hbm.at[pl.ds(i0, n_chunks)], idxbuf)

    descs = {}

    def start_load(i):
        descs["x", i] = pltpu.async_copy(
            x_hbm.at[pl.ds(g0 + i * _CHG, _CHG)], xbuf.at[i % 3],
            sem_x[i % 3])

    def start_gather_add(i):
        descs["g", i] = pltpu.async_copy(
            pos_hbm.at[idxbuf.at[i]], xbuf.at[i % 3], sem_g[i % 3], add=True)

    def start_store(i):
        descs["st", i] = pltpu.async_copy(
            xbuf.at[i % 3], out_hbm.at[pl.ds(g0 + i * _CHG, _CHG)],
            sem_st[i % 3])

    for i in range(n_chunks + 2):
        if i < n_chunks:
            if i >= 3:
                descs["st", i - 3].wait()
            start_load(i)
        if 0 <= i - 1 < n_chunks:
            descs["x", i - 1].wait()
            start_gather_add(i - 1)
        if 0 <= i - 2 < n_chunks:
            descs["g", i - 2].wait()
            start_store(i - 2)
    for i in range(max(0, n_chunks - 3), n_chunks):
        descs["st", i].wait()


def kernel(x, pos_table):
    B, L, D = x.shape
    sl = _G * D // _MINOR
    n_total_g = B * L // _G
    n_pos_g = L // _G
    x5 = x.reshape(n_total_g, sl, _MINOR)
    pos5 = pos_table.reshape(pos_table.shape[0] // _G, sl, _MINOR)
    idx_all = (jnp.arange(n_total_g, dtype=jnp.int32) % n_pos_g).reshape(
        n_total_g // _CHG, _CHG)

    mesh = plsc.VectorSubcoreMesh(core_axis_name="c", subcore_axis_name="s")
    n_chunks_w = (n_total_g // _NW) // _CHG
    sc = pl.kernel(
        functools.partial(_sc_body, n_total_g, n_pos_g, sl),
        out_type=jax.ShapeDtypeStruct((n_total_g, sl, _MINOR), jnp.float32),
        mesh=mesh,
        scratch_types=[
            pltpu.VMEM((3, _CHG, sl, _MINOR), jnp.float32),
            pltpu.VMEM((n_chunks_w, _CHG), jnp.int32),
            pltpu.SemaphoreType.DMA,
            pltpu.SemaphoreType.DMA,
            pltpu.SemaphoreType.DMA,
            pltpu.SemaphoreType.DMA,
            pltpu.SemaphoreType.DMA,
            pltpu.SemaphoreType.DMA,
            pltpu.SemaphoreType.DMA,
            pltpu.SemaphoreType.DMA,
            pltpu.SemaphoreType.DMA,
        ],
    )
    out = sc(x5, pos5, idx_all)
    return out.reshape(B, L, D)


# speedup vs baseline: 3.3006x; 3.3006x over previous
import functools

import jax
import jax.numpy as jnp
from jax import lax
from jax.experimental import pallas as pl
from jax.experimental.pallas import tpu as pltpu
from jax.experimental.pallas import tpu_sc as plsc

_LANES = 16
_NW = 32
_CH = 16


def _sc_body(B, L, D, x_raw, pos_hbm, out_raw, xbuf, pbuf,
             sx0, sx1, sx2, st0, st1, st2, sp):
    x_hbm = x_raw.reshape(B * L, D)
    out_hbm = out_raw.reshape(B * L, D)
    cid = lax.axis_index("c")
    sid = lax.axis_index("s")
    wid = sid * 2 + cid
    pos_per_w = L // _NW
    n_chunks = pos_per_w // _CH
    pos_lo = wid * pos_per_w

    sem_x = [sx0, sx1, sx2]
    sem_st = [st0, st1, st2]
    steps = [(c, b) for c in range(n_chunks) for b in range(B)]
    n_steps = len(steps)

    def x_row(c, b):
        return b * L + pos_lo + c * _CH

    descs = {}

    def start_load_x(i):
        c, b = steps[i]
        descs["x", i] = pltpu.async_copy(
            x_hbm.at[pl.ds(x_row(c, b), _CH), :], xbuf.at[i % 3],
            sem_x[i % 3])

    def start_load_p(c):
        descs["p", c] = pltpu.async_copy(
            pos_hbm.at[pl.ds(pos_lo + c * _CH, _CH), :], pbuf.at[c % 2], sp)

    def start_store(i):
        c, b = steps[i]
        descs["st", i] = pltpu.async_copy(
            xbuf.at[i % 3], out_hbm.at[pl.ds(x_row(c, b), _CH), :],
            sem_st[i % 3])

    start_load_p(0)
    start_load_x(0)
    for i, (c, b) in enumerate(steps):
        slot = i % 3
        if b == 0:
            descs["p", c].wait()
            if c + 1 < n_chunks:
                start_load_p(c + 1)
        if i + 1 < n_steps:
            if i + 1 >= 3:
                descs["st", i - 2].wait()
            start_load_x(i + 1)
        descs["x", i].wait()

        n_sl = D // _LANES

        def add_slice(t, slot=slot, pslot=c % 2):
            r = t // n_sl
            sl = pl.ds(lax.rem(t, n_sl) * _LANES, _LANES)
            plsc.addupdate(xbuf.at[slot, r, sl], pbuf[pslot, r, sl])

        plsc.parallel_loop(0, _CH * n_sl, 1, unroll=8)(add_slice)
        start_store(i)
    for i in range(max(0, n_steps - 3), n_steps):
        descs["st", i].wait()


def kernel(x, pos_table):
    B, L, D = x.shape

    mesh = plsc.VectorSubcoreMesh(core_axis_name="c", subcore_axis_name="s")
    sc = pl.kernel(
        functools.partial(_sc_body, B, L, D),
        out_type=jax.ShapeDtypeStruct((B, L, D), jnp.float32),
        mesh=mesh,
        scratch_types=[
            pltpu.VMEM((3, _CH, D), jnp.float32),
            pltpu.VMEM((2, _CH, D), jnp.float32),
            pltpu.SemaphoreType.DMA,
            pltpu.SemaphoreType.DMA,
            pltpu.SemaphoreType.DMA,
            pltpu.SemaphoreType.DMA,
            pltpu.SemaphoreType.DMA,
            pltpu.SemaphoreType.DMA,
            pltpu.SemaphoreType.DMA,
        ],
    )
    return sc(x, pos_table)
